# baseline (speedup 1.0000x reference)
import functools

import jax
import jax.numpy as jnp
from jax import lax
from jax.experimental import pallas as pl
from jax.experimental.pallas import tpu as pltpu
from jax.experimental.pallas import tpu_sc as plsc

N = 10000
E = 320000
D = 128

NC = 2
NS = 16
NW = NC * NS
EPT = E // NW
CH = 40
NCHUNK = EPT // CH
NROWCH = N // CH
RRB = (NROWCH + NS - 1) // NS

RB = 2000
GRID = N // RB


def _zero_rows(ref, rows):
    def zrow(i, _):
        for j in range(D // 16):
            ref[i, pl.ds(j * 16, 16)] = jnp.zeros((16,), jnp.float32)
        return 0
    lax.fori_loop(0, rows, zrow, 0)


def _acc_rr_copy(s, body):
    def step(k, _):
        idx = s + NS * k
        @pl.when(idx < NROWCH)
        def _():
            body(idx)
        return 0
    lax.fori_loop(0, RRB, step, 0)


def _edge_body(xl_hbm, bg_hbm, srcr_hbm, dstr_hbm, out_hbm,
               src_full, dst_full, dsl_a, dsl_b, xj_a, bg_a, xj_b, bg_b,
               acc_sh, gsa, gsb, ssa, ssb):
    c = lax.axis_index("c")
    s = lax.axis_index("s")
    wid = s * NC + c

    pltpu.sync_copy(srcr_hbm.at[wid], src_full)
    pltpu.sync_copy(dstr_hbm.at[wid], dst_full)

    _zero_rows(xj_a, CH)
    _acc_rr_copy(s, lambda idx: pltpu.sync_copy(
        xj_a, acc_sh.at[pl.ds(idx * CH, CH)]))
    plsc.subcore_barrier()

    def issue_gathers(k, xv, bv, sem):
        pltpu.async_copy(xl_hbm.at[src_full.at[pl.ds(k * CH, CH)]], xv, sem)
        pltpu.async_copy(bg_hbm.at[dst_full.at[pl.ds(k * CH, CH)]], bv, sem)

    def wait_gathers(k, xv, bv, sem):
        pltpu.make_async_copy(xl_hbm.at[src_full.at[pl.ds(k * CH, CH)]], xv,
                              sem).wait()
        pltpu.make_async_copy(bg_hbm.at[dst_full.at[pl.ds(k * CH, CH)]], bv,
                              sem).wait()

    def fill_slot(k, slot_ref):
        for o in (0, 16, CH - 16):
            slot_ref[pl.ds(o, 16)] = dst_full[pl.ds(k * CH + o, 16)]

    def compute(xv, bv):
        hi_mask = jnp.int32(-65536)

        @plsc.parallel_loop(0, CH, unroll=4)
        def edge(e):
            for g in range(D // 32):
                vb = bv[e, pl.ds(16 * g, 16)]
                vg = bv[e, pl.ds(64 + 16 * g, 16)]
                blo = lax.bitcast_convert_type(lax.shift_left(vb, 16), jnp.float32)
                bhi = lax.bitcast_convert_type(jnp.bitwise_and(vb, hi_mask), jnp.float32)
                glo = lax.bitcast_convert_type(lax.shift_left(vg, 16), jnp.float32)
                ghi = lax.bitcast_convert_type(jnp.bitwise_and(vg, hi_mask), jnp.float32)
                xlo = xv[e, pl.ds(16 * g, 16)]
                xhi = xv[e, pl.ds(64 + 16 * g, 16)]
                xv[e, pl.ds(16 * g, 16)] = jnp.maximum(glo * xlo + blo, 0.0)
                xv[e, pl.ds(64 + 16 * g, 16)] = jnp.maximum(ghi * xhi + bhi, 0.0)

    def issue_scatter(k, xv, dv, sem):
        fill_slot(k, dv)
        pltpu.async_copy(xv, acc_sh.at[dv], sem, add=True)

    def wait_scatter(xv, dv, sem):
        pltpu.make_async_copy(xv, acc_sh.at[dv], sem).wait()

    issue_gathers(0, xj_b, bg_b, gsb)
    wait_gathers(0, xj_b, bg_b, gsb)
    compute(xj_b, bg_b)
    fill_slot(0, dsl_b)
    pltpu.sync_copy(xj_b, acc_sh.at[dsl_b], add=True)
    issue_gathers(1, xj_a, bg_a, gsa)

    npair = (NCHUNK - 1) // 2
    leftover = (NCHUNK - 1) % 2

    def body(m, _):
        @pl.when(m >= 1)
        def _():
            wait_scatter(xj_b, dsl_b, ssb)
        issue_gathers(2 * m + 2, xj_b, bg_b, gsb)

        wait_gathers(2 * m + 1, xj_a, bg_a, gsa)
        compute(xj_a, bg_a)
        issue_scatter(2 * m + 1, xj_a, dsl_a, ssa)

        wait_gathers(2 * m + 2, xj_b, bg_b, gsb)

        @pl.when(2 * m + 3 < NCHUNK)
        def _():
            wait_scatter(xj_a, dsl_a, ssa)
            issue_gathers(2 * m + 3, xj_a, bg_a, gsa)

        compute(xj_b, bg_b)
        issue_scatter(2 * m + 2, xj_b, dsl_b, ssb)
        return 0
    lax.fori_loop(0, npair, body, 0)

    if leftover:
        wait_gathers(NCHUNK - 1, xj_a, bg_a, gsa)
        compute(xj_a, bg_a)
        fill_slot(NCHUNK - 1, dsl_a)
        pltpu.sync_copy(xj_a, acc_sh.at[dsl_a], add=True)
        wait_scatter(xj_b, dsl_b, ssb)
    else:
        wait_scatter(xj_a, dsl_a, ssa)
        wait_scatter(xj_b, dsl_b, ssb)

    plsc.subcore_barrier()

    _acc_rr_copy(s, lambda idx: pltpu.sync_copy(
        acc_sh.at[pl.ds(idx * CH, CH)], out_hbm.at[c, pl.ds(idx * CH, CH)]))


_edge_call = functools.partial(
    pl.kernel,
    mesh=plsc.VectorSubcoreMesh(core_axis_name="c", subcore_axis_name="s"),
    out_type=jax.ShapeDtypeStruct((NC, N, D), jnp.float32),
    scratch_types=[
        pltpu.VMEM((EPT,), jnp.int32),
        pltpu.VMEM((EPT,), jnp.int32),
        pltpu.VMEM((CH,), jnp.int32),
        pltpu.VMEM((CH,), jnp.int32),
        pltpu.VMEM((CH, D), jnp.float32),
        pltpu.VMEM((CH, D), jnp.int32),
        pltpu.VMEM((CH, D), jnp.float32),
        pltpu.VMEM((CH, D), jnp.int32),
        pltpu.VMEM_SHARED((N, D), jnp.float32),
        pltpu.SemaphoreType.DMA,
        pltpu.SemaphoreType.DMA,
        pltpu.SemaphoreType.DMA,
        pltpu.SemaphoreType.DMA,
    ],
)(_edge_body)


def _deg_body(dstr_hbm, out_hbm, dst_full, ones_v, acc_sh, sem):
    c = lax.axis_index("c")
    s = lax.axis_index("s")
    wid = s * NC + c

    pltpu.sync_copy(dstr_hbm.at[wid], dst_full)

    _zero_rows(ones_v, CH)
    _acc_rr_copy(s, lambda idx: pltpu.sync_copy(
        ones_v, acc_sh.at[pl.ds(idx * CH, CH)]))

    def orow(i, _):
        for j in range(D // 16):
            ones_v[i, pl.ds(j * 16, 16)] = jnp.ones((16,), jnp.float32)
        return 0
    lax.fori_loop(0, CH, orow, 0)
    plsc.subcore_barrier()

    LAG = 8

    def chunk(k, _):
        pltpu.async_copy(ones_v, acc_sh.at[dst_full.at[k]], sem, add=True)

        @pl.when(k >= LAG)
        def _():
            pltpu.make_async_copy(ones_v, acc_sh.at[dst_full.at[k - LAG]],
                                  sem).wait()
        return 0
    lax.fori_loop(0, NCHUNK, chunk, 0)
    for i in range(LAG):
        pltpu.make_async_copy(ones_v, acc_sh.at[dst_full.at[NCHUNK - LAG + i]],
                              sem).wait()

    plsc.subcore_barrier()

    _acc_rr_copy(s, lambda idx: pltpu.sync_copy(
        acc_sh.at[pl.ds(idx * CH, CH)], out_hbm.at[c, pl.ds(idx * CH, CH)]))


_deg_call = functools.partial(
    pl.kernel,
    mesh=plsc.VectorSubcoreMesh(core_axis_name="c", subcore_axis_name="s"),
    out_type=jax.ShapeDtypeStruct((NC, N, D), jnp.float32),
    scratch_types=[
        pltpu.VMEM((NCHUNK, CH), jnp.int32),
        pltpu.VMEM((CH, D), jnp.float32),
        pltpu.VMEM_SHARED((N, D), jnp.float32),
        pltpu.SemaphoreType.DMA,
    ],
)(_deg_body)


def _pack_half(t):
    u = lax.bitcast_convert_type(t.astype(jnp.bfloat16), jnp.uint16)
    u = u.astype(jnp.uint32)
    return lax.bitcast_convert_type(u[:, :64] | (u[:, 64:] << 16), jnp.int32)


def _dense(h, Wl, Wf, bf, Wls, Wfs):
    xl = jnp.dot(h, Wl, preferred_element_type=jnp.float32)
    bg = jnp.dot(h, Wf, preferred_element_type=jnp.float32) + bf
    bgs = jnp.dot(h, Wfs, preferred_element_type=jnp.float32)
    lin = jnp.dot(h, Wls, preferred_element_type=jnp.float32)
    out = jnp.maximum(bgs[:, D:] * lin + bgs[:, :D], 0.0)
    bgp = jnp.concatenate([_pack_half(bg[:, :D]), _pack_half(bg[:, D:])],
                          axis=1)
    return xl, bgp, out


def _pre_body(x_ref, wl_ref, wf_ref, bf_ref, wls_ref, wfs_ref,
              xl_ref, bg_ref, out_ref):
    xl, bg, out = _dense(x_ref[...], wl_ref[...], wf_ref[...], bf_ref[...],
                         wls_ref[...], wfs_ref[...])
    xl_ref[...] = xl
    bg_ref[...] = bg
    out_ref[...] = out


def _combine(agg_ref, deg_ref, outp_ref):
    a = agg_ref[0] + agg_ref[1]
    deg = deg_ref[0, :, :1] + deg_ref[1, :, :1]
    inv = 1.0 / jnp.maximum(deg, 1.0)
    return jnp.maximum(outp_ref[...] + a * inv, 0.0)


def _mid_body(agg_ref, deg_ref, outp_ref, wl_ref, wf_ref, bf_ref, wls_ref,
              wfs_ref, xl_ref, bg_ref, out_ref):
    h = _combine(agg_ref, deg_ref, outp_ref)
    xl, bg, out = _dense(h, wl_ref[...], wf_ref[...], bf_ref[...],
                         wls_ref[...], wfs_ref[...])
    xl_ref[...] = xl
    bg_ref[...] = bg
    out_ref[...] = out


def _post_body(agg_ref, deg_ref, outp_ref, y_ref):
    h = _combine(agg_ref, deg_ref, outp_ref)
    m = jnp.max(h, axis=-1, keepdims=True)
    ex = jnp.exp(h - m)
    y_ref[...] = h - (jnp.log(jnp.sum(ex, axis=-1, keepdims=True)) + m)


def _row_spec(w):
    return pl.BlockSpec((RB, w), lambda i: (i, 0))


def _full_spec(shape):
    nd = len(shape)
    return pl.BlockSpec(shape, lambda i: (0,) * nd)


_W_SPECS = [
    _full_spec((D, D)),
    _full_spec((D, 2 * D)),
    _full_spec((1, 2 * D)),
    _full_spec((D, D)),
    _full_spec((D, 2 * D)),
]

_DENSE_OUT = [
    jax.ShapeDtypeStruct((N, D), jnp.float32),
    jax.ShapeDtypeStruct((N, D), jnp.int32),
    jax.ShapeDtypeStruct((N, D), jnp.float32),
]

_pre_call = pl.pallas_call(
    _pre_body,
    grid=(GRID,),
    in_specs=[_row_spec(D)] + _W_SPECS,
    out_specs=[_row_spec(D), _row_spec(D), _row_spec(D)],
    out_shape=_DENSE_OUT,
)

_AGG_SPEC = pl.BlockSpec((NC, RB, D), lambda i: (0, i, 0))

_mid_call = pl.pallas_call(
    _mid_body,
    grid=(GRID,),
    in_specs=[_AGG_SPEC, _AGG_SPEC, _row_spec(D)] + _W_SPECS,
    out_specs=[_row_spec(D), _row_spec(D), _row_spec(D)],
    out_shape=_DENSE_OUT,
)

_post_call = pl.pallas_call(
    _post_body,
    grid=(GRID,),
    in_specs=[_AGG_SPEC, _AGG_SPEC, _row_spec(D)],
    out_specs=_row_spec(D),
    out_shape=jax.ShapeDtypeStruct((N, D), jnp.float32),
)


def kernel(x, edge_index, W_lin, W_film, b_film, W_skip, W_film_skip):
    srcr = edge_index[0].reshape(NW, EPT)
    dstr = edge_index[1].reshape(NW, EPT)

    deg2 = _deg_call(edge_index[1].reshape(NW, NCHUNK, CH))
    xl, bg, out = _pre_call(x, W_lin[0], W_film[0], b_film[0].reshape(1, -1),
                            W_skip[0], W_film_skip[0])
    agg = _edge_call(xl, bg, srcr, dstr)
    for l in range(1, 3):
        xl, bg, out = _mid_call(agg, deg2, out, W_lin[l], W_film[l],
                                b_film[l].reshape(1, -1), W_skip[l],
                                W_film_skip[l])
        agg = _edge_call(xl, bg, srcr, dstr)
    return _post_call(agg, deg2, out)


# speedup vs baseline: 1.1195x; 1.1195x over previous
import functools

import jax
import jax.numpy as jnp
from jax import lax
from jax.experimental import pallas as pl
from jax.experimental.pallas import tpu as pltpu
from jax.experimental.pallas import tpu_sc as plsc

N = 10000
E = 320000
D = 128

NC = 2
NS = 16
NW = NC * NS
EPT = E // NW
CH = 40
NCHUNK = EPT // CH
NROWCH = N // CH
RRB = (NROWCH + NS - 1) // NS

RB = 2000
GRID = N // RB


def _zero_rows(ref, rows):
    def zrow(i, _):
        for j in range(D // 16):
            ref[i, pl.ds(j * 16, 16)] = jnp.zeros((16,), jnp.float32)
        return 0
    lax.fori_loop(0, rows, zrow, 0)


def _acc_rr_copy(s, body):
    def step(k, _):
        idx = s + NS * k
        @pl.when(idx < NROWCH)
        def _():
            body(idx)
        return 0
    lax.fori_loop(0, RRB, step, 0)


def _edge_body(xl_hbm, bg_hbm, packed_hbm, out_hbm,
               packed_full, ssl_a, ssl_b, ssl_c, dsl_a, dsl_b, dsl_c,
               xj_a, bg_a, xj_b, bg_b, xj_c, bg_c,
               acc_sh, gsa, gsb, gsc, ssa, ssb, ssc):
    c = lax.axis_index("c")
    s = lax.axis_index("s")
    wid = s * NC + c

    pltpu.sync_copy(packed_hbm.at[wid], packed_full)

    _zero_rows(xj_a, CH)
    _acc_rr_copy(s, lambda idx: pltpu.sync_copy(
        xj_a, acc_sh.at[pl.ds(idx * CH, CH)]))
    plsc.subcore_barrier()

    def fill_slots(k, sv, dv):
        lo_mask = jnp.int32(0xFFFF)
        for o in (0, 16, CH - 16):
            p = packed_full[pl.ds(k * CH + o, 16)]
            sv[pl.ds(o, 16)] = jnp.bitwise_and(p, lo_mask)
            dv[pl.ds(o, 16)] = lax.shift_right_logical(p, 16)

    def issue_gathers(sv, dv, xv, bv, sem):
        pltpu.async_copy(xl_hbm.at[sv], xv, sem)
        pltpu.async_copy(bg_hbm.at[dv], bv, sem)

    def wait_gathers(sv, dv, xv, bv, sem):
        pltpu.make_async_copy(xl_hbm.at[sv], xv, sem).wait()
        pltpu.make_async_copy(bg_hbm.at[dv], bv, sem).wait()

    def compute(xv, bv):
        hi_mask = jnp.int32(-65536)

        @plsc.parallel_loop(0, CH, unroll=2)
        def edge(e):
            for g in range(D // 32):
                vb = bv[e, pl.ds(16 * g, 16)]
                vg = bv[e, pl.ds(64 + 16 * g, 16)]
                blo = lax.bitcast_convert_type(lax.shift_left(vb, 16), jnp.float32)
                bhi = lax.bitcast_convert_type(jnp.bitwise_and(vb, hi_mask), jnp.float32)
                glo = lax.bitcast_convert_type(lax.shift_left(vg, 16), jnp.float32)
                ghi = lax.bitcast_convert_type(jnp.bitwise_and(vg, hi_mask), jnp.float32)
                xlo = xv[e, pl.ds(16 * g, 16)]
                xhi = xv[e, pl.ds(64 + 16 * g, 16)]
                xv[e, pl.ds(16 * g, 16)] = jnp.maximum(glo * xlo + blo, 0.0)
                xv[e, pl.ds(64 + 16 * g, 16)] = jnp.maximum(ghi * xhi + bhi, 0.0)

    def issue_scatter(xv, dv, sem):
        pltpu.async_copy(xv, acc_sh.at[dv], sem, add=True)

    def wait_scatter(xv, dv, sem):
        pltpu.make_async_copy(xv, acc_sh.at[dv], sem).wait()

    slots = [(xj_a, bg_a, ssl_a, dsl_a, gsa, ssa),
             (xj_b, bg_b, ssl_b, dsl_b, gsb, ssb),
             (xj_c, bg_c, ssl_c, dsl_c, gsc, ssc)]

    assert (NCHUNK - 1) % 3 == 0
    ntri = (NCHUNK - 1) // 3

    fill_slots(0, ssl_c, dsl_c)
    issue_gathers(ssl_c, dsl_c, xj_c, bg_c, gsc)
    wait_gathers(ssl_c, dsl_c, xj_c, bg_c, gsc)
    compute(xj_c, bg_c)
    pltpu.sync_copy(xj_c, acc_sh.at[dsl_c], add=True)
    fill_slots(1, ssl_a, dsl_a)
    issue_gathers(ssl_a, dsl_a, xj_a, bg_a, gsa)
    fill_slots(2, ssl_b, dsl_b)
    issue_gathers(ssl_b, dsl_b, xj_b, bg_b, gsb)

    def body(m, _):
        for j in range(3):
            k = 3 * m + 1 + j
            xv, bv, sv, dv, gs, ss = slots[j % 3]
            pxv, pbv, psv, pdv, pgs, pss = slots[(j + 2) % 3]

            wait_gathers(sv, dv, xv, bv, gs)
            compute(xv, bv)
            issue_scatter(xv, dv, ss)

            if j == 0:
                @pl.when(m >= 1)
                def _():
                    wait_scatter(pxv, pdv, pss)
            else:
                wait_scatter(pxv, pdv, pss)

            @pl.when(k + 2 < NCHUNK)
            def _():
                fill_slots(k + 2, psv, pdv)
                issue_gathers(psv, pdv, pxv, pbv, pgs)
        return 0
    lax.fori_loop(0, ntri, body, 0)

    wait_scatter(xj_c, dsl_c, ssc)

    plsc.subcore_barrier()

    _acc_rr_copy(s, lambda idx: pltpu.sync_copy(
        acc_sh.at[pl.ds(idx * CH, CH)], out_hbm.at[c, pl.ds(idx * CH, CH)]))


_edge_call = functools.partial(
    pl.kernel,
    mesh=plsc.VectorSubcoreMesh(core_axis_name="c", subcore_axis_name="s"),
    out_type=jax.ShapeDtypeStruct((NC, N, D), jnp.float32),
    scratch_types=[
        pltpu.VMEM((EPT,), jnp.int32),
        pltpu.VMEM((CH,), jnp.int32),
        pltpu.VMEM((CH,), jnp.int32),
        pltpu.VMEM((CH,), jnp.int32),
        pltpu.VMEM((CH,), jnp.int32),
        pltpu.VMEM((CH,), jnp.int32),
        pltpu.VMEM((CH,), jnp.int32),
        pltpu.VMEM((CH, D), jnp.float32),
        pltpu.VMEM((CH, D), jnp.int32),
        pltpu.VMEM((CH, D), jnp.float32),
        pltpu.VMEM((CH, D), jnp.int32),
        pltpu.VMEM((CH, D), jnp.float32),
        pltpu.VMEM((CH, D), jnp.int32),
        pltpu.VMEM_SHARED((N, D), jnp.float32),
        pltpu.SemaphoreType.DMA,
        pltpu.SemaphoreType.DMA,
        pltpu.SemaphoreType.DMA,
        pltpu.SemaphoreType.DMA,
        pltpu.SemaphoreType.DMA,
        pltpu.SemaphoreType.DMA,
    ],
)(_edge_body)


def _deg_body(dstr_hbm, out_hbm, dst_full, ones_v, acc_sh, sem):
    c = lax.axis_index("c")
    s = lax.axis_index("s")
    wid = s * NC + c

    pltpu.sync_copy(dstr_hbm.at[wid], dst_full)

    _zero_rows(ones_v, CH)
    _acc_rr_copy(s, lambda idx: pltpu.sync_copy(
        ones_v, acc_sh.at[pl.ds(idx * CH, CH)]))

    def orow(i, _):
        for j in range(D // 16):
            ones_v[i, pl.ds(j * 16, 16)] = jnp.ones((16,), jnp.float32)
        return 0
    lax.fori_loop(0, CH, orow, 0)
    plsc.subcore_barrier()

    LAG = 8

    def chunk(k, _):
        pltpu.async_copy(ones_v, acc_sh.at[dst_full.at[k]], sem, add=True)

        @pl.when(k >= LAG)
        def _():
            pltpu.make_async_copy(ones_v, acc_sh.at[dst_full.at[k - LAG]],
                                  sem).wait()
        return 0
    lax.fori_loop(0, NCHUNK, chunk, 0)
    for i in range(LAG):
        pltpu.make_async_copy(ones_v, acc_sh.at[dst_full.at[NCHUNK - LAG + i]],
                              sem).wait()

    plsc.subcore_barrier()

    _acc_rr_copy(s, lambda idx: pltpu.sync_copy(
        acc_sh.at[pl.ds(idx * CH, CH)], out_hbm.at[c, pl.ds(idx * CH, CH)]))


_deg_call = functools.partial(
    pl.kernel,
    mesh=plsc.VectorSubcoreMesh(core_axis_name="c", subcore_axis_name="s"),
    out_type=jax.ShapeDtypeStruct((NC, N, D), jnp.float32),
    scratch_types=[
        pltpu.VMEM((NCHUNK, CH), jnp.int32),
        pltpu.VMEM((CH, D), jnp.float32),
        pltpu.VMEM_SHARED((N, D), jnp.float32),
        pltpu.SemaphoreType.DMA,
    ],
)(_deg_body)


def _pack_half(t):
    u = lax.bitcast_convert_type(t.astype(jnp.bfloat16), jnp.uint16)
    u = u.astype(jnp.uint32)
    return lax.bitcast_convert_type(u[:, :64] | (u[:, 64:] << 16), jnp.int32)


def _dense(h, Wl, Wf, bf, Wls, Wfs):
    xl = jnp.dot(h, Wl, preferred_element_type=jnp.float32)
    bg = jnp.dot(h, Wf, preferred_element_type=jnp.float32) + bf
    bgs = jnp.dot(h, Wfs, preferred_element_type=jnp.float32)
    lin = jnp.dot(h, Wls, preferred_element_type=jnp.float32)
    out = jnp.maximum(bgs[:, D:] * lin + bgs[:, :D], 0.0)
    bgp = jnp.concatenate([_pack_half(bg[:, :D]), _pack_half(bg[:, D:])],
                          axis=1)
    return xl, bgp, out


def _pre_body(x_ref, wl_ref, wf_ref, bf_ref, wls_ref, wfs_ref,
              xl_ref, bg_ref, out_ref):
    xl, bg, out = _dense(x_ref[...], wl_ref[...], wf_ref[...], bf_ref[...],
                         wls_ref[...], wfs_ref[...])
    xl_ref[...] = xl
    bg_ref[...] = bg
    out_ref[...] = out


def _combine(agg_ref, deg_ref, outp_ref):
    a = agg_ref[0] + agg_ref[1]
    deg = deg_ref[0, :, :1] + deg_ref[1, :, :1]
    inv = 1.0 / jnp.maximum(deg, 1.0)
    return jnp.maximum(outp_ref[...] + a * inv, 0.0)


def _mid_body(agg_ref, deg_ref, outp_ref, wl_ref, wf_ref, bf_ref, wls_ref,
              wfs_ref, xl_ref, bg_ref, out_ref):
    h = _combine(agg_ref, deg_ref, outp_ref)
    xl, bg, out = _dense(h, wl_ref[...], wf_ref[...], bf_ref[...],
                         wls_ref[...], wfs_ref[...])
    xl_ref[...] = xl
    bg_ref[...] = bg
    out_ref[...] = out


def _post_body(agg_ref, deg_ref, outp_ref, y_ref):
    h = _combine(agg_ref, deg_ref, outp_ref)
    m = jnp.max(h, axis=-1, keepdims=True)
    ex = jnp.exp(h - m)
    y_ref[...] = h - (jnp.log(jnp.sum(ex, axis=-1, keepdims=True)) + m)


def _row_spec(w):
    return pl.BlockSpec((RB, w), lambda i: (i, 0))


def _full_spec(shape):
    nd = len(shape)
    return pl.BlockSpec(shape, lambda i: (0,) * nd)


_W_SPECS = [
    _full_spec((D, D)),
    _full_spec((D, 2 * D)),
    _full_spec((1, 2 * D)),
    _full_spec((D, D)),
    _full_spec((D, 2 * D)),
]

_DENSE_OUT = [
    jax.ShapeDtypeStruct((N, D), jnp.float32),
    jax.ShapeDtypeStruct((N, D), jnp.int32),
    jax.ShapeDtypeStruct((N, D), jnp.float32),
]

_pre_call = pl.pallas_call(
    _pre_body,
    grid=(GRID,),
    in_specs=[_row_spec(D)] + _W_SPECS,
    out_specs=[_row_spec(D), _row_spec(D), _row_spec(D)],
    out_shape=_DENSE_OUT,
)

_AGG_SPEC = pl.BlockSpec((NC, RB, D), lambda i: (0, i, 0))

_mid_call = pl.pallas_call(
    _mid_body,
    grid=(GRID,),
    in_specs=[_AGG_SPEC, _AGG_SPEC, _row_spec(D)] + _W_SPECS,
    out_specs=[_row_spec(D), _row_spec(D), _row_spec(D)],
    out_shape=_DENSE_OUT,
)

_post_call = pl.pallas_call(
    _post_body,
    grid=(GRID,),
    in_specs=[_AGG_SPEC, _AGG_SPEC, _row_spec(D)],
    out_specs=_row_spec(D),
    out_shape=jax.ShapeDtypeStruct((N, D), jnp.float32),
)


def kernel(x, edge_index, W_lin, W_film, b_film, W_skip, W_film_skip):
    packed = (edge_index[0] | (edge_index[1] << 16)).reshape(NW, EPT)

    deg2 = _deg_call(edge_index[1].reshape(NW, NCHUNK, CH))
    xl, bg, out = _pre_call(x, W_lin[0], W_film[0], b_film[0].reshape(1, -1),
                            W_skip[0], W_film_skip[0])
    agg = _edge_call(xl, bg, packed)
    for l in range(1, 3):
        xl, bg, out = _mid_call(agg, deg2, out, W_lin[l], W_film[l],
                                b_film[l].reshape(1, -1), W_skip[l],
                                W_film_skip[l])
        agg = _edge_call(xl, bg, packed)
    return _post_call(agg, deg2, out)


# speedup vs baseline: 1.2698x; 1.1342x over previous
import functools

import jax
import jax.numpy as jnp
from jax import lax
from jax.experimental import pallas as pl
from jax.experimental.pallas import tpu as pltpu
from jax.experimental.pallas import tpu_sc as plsc

N = 10000
E = 320000
D = 128

NC = 2
NS = 16
NW = NC * NS
EPT = E // NW
CH = 40
NCHUNK = EPT // CH
NROWCH = N // CH
RRB = (NROWCH + NS - 1) // NS

RB = 2000
GRID = N // RB


def _zero_rows(ref, rows):
    def zrow(i, _):
        for j in range(D // 16):
            ref[i, pl.ds(j * 16, 16)] = jnp.zeros((16,), jnp.float32)
        return 0
    lax.fori_loop(0, rows, zrow, 0)


def _acc_rr_copy(s, body):
    def step(k, _):
        idx = s + NS * k
        @pl.when(idx < NROWCH)
        def _():
            body(idx)
        return 0
    lax.fori_loop(0, RRB, step, 0)


def _edge_body(xl_hbm, bg_hbm, packed_hbm, out_hbm,
               packed_full, ssl_a, ssl_b, ssl_c, dsl_a, dsl_b, dsl_c,
               xj_a, bg_a, xj_b, bg_b, xj_c, bg_c,
               acc_sh, gsa, gsb, gsc, ssa, ssb, ssc):
    c = lax.axis_index("c")
    s = lax.axis_index("s")
    wid = s * NC + c

    pltpu.sync_copy(packed_hbm.at[wid], packed_full)

    _zero_rows(xj_a, CH)
    _acc_rr_copy(s, lambda idx: pltpu.sync_copy(
        xj_a, acc_sh.at[pl.ds(idx * CH, CH)]))
    plsc.subcore_barrier()

    def fill_slots(k, sv, dv):
        lo_mask = jnp.int32(0xFFFF)
        for o in (0, 16, CH - 16):
            p = packed_full[pl.ds(k * CH + o, 16)]
            sv[pl.ds(o, 16)] = jnp.bitwise_and(p, lo_mask)
            dv[pl.ds(o, 16)] = lax.shift_right_logical(p, 16)

    def issue_gathers(sv, dv, xv, bv, sem):
        pltpu.async_copy(xl_hbm.at[sv], xv, sem)
        pltpu.async_copy(bg_hbm.at[dv], bv, sem)

    def wait_gathers(sv, dv, xv, bv, sem):
        pltpu.make_async_copy(xl_hbm.at[sv], xv, sem).wait()
        pltpu.make_async_copy(bg_hbm.at[dv], bv, sem).wait()

    def compute(xv, bv):
        hi_mask = jnp.int32(-65536)

        @plsc.parallel_loop(0, CH, unroll=2)
        def edge(e):
            for g in range(D // 32):
                vb = bv[e, pl.ds(16 * g, 16)]
                vg = bv[e, pl.ds(64 + 16 * g, 16)]
                blo = lax.bitcast_convert_type(lax.shift_left(vb, 16), jnp.float32)
                bhi = lax.bitcast_convert_type(jnp.bitwise_and(vb, hi_mask), jnp.float32)
                glo = lax.bitcast_convert_type(lax.shift_left(vg, 16), jnp.float32)
                ghi = lax.bitcast_convert_type(jnp.bitwise_and(vg, hi_mask), jnp.float32)
                xlo = xv[e, pl.ds(16 * g, 16)]
                xhi = xv[e, pl.ds(64 + 16 * g, 16)]
                xv[e, pl.ds(16 * g, 16)] = jnp.maximum(glo * xlo + blo, 0.0)
                xv[e, pl.ds(64 + 16 * g, 16)] = jnp.maximum(ghi * xhi + bhi, 0.0)

    def issue_scatter(xv, dv, sem):
        pltpu.async_copy(xv, acc_sh.at[dv], sem, add=True)

    def wait_scatter(xv, dv, sem):
        pltpu.make_async_copy(xv, acc_sh.at[dv], sem).wait()

    slots = [(xj_a, bg_a, ssl_a, dsl_a, gsa, ssa),
             (xj_b, bg_b, ssl_b, dsl_b, gsb, ssb),
             (xj_c, bg_c, ssl_c, dsl_c, gsc, ssc)]

    assert (NCHUNK - 1) % 3 == 0
    ntri = (NCHUNK - 1) // 3

    fill_slots(0, ssl_c, dsl_c)
    issue_gathers(ssl_c, dsl_c, xj_c, bg_c, gsc)
    wait_gathers(ssl_c, dsl_c, xj_c, bg_c, gsc)
    compute(xj_c, bg_c)
    pltpu.sync_copy(xj_c, acc_sh.at[dsl_c], add=True)
    fill_slots(1, ssl_a, dsl_a)
    issue_gathers(ssl_a, dsl_a, xj_a, bg_a, gsa)
    fill_slots(2, ssl_b, dsl_b)
    issue_gathers(ssl_b, dsl_b, xj_b, bg_b, gsb)

    def body(m, _):
        for j in range(3):
            k = 3 * m + 1 + j
            xv, bv, sv, dv, gs, ss = slots[j % 3]
            pxv, pbv, psv, pdv, pgs, pss = slots[(j + 2) % 3]

            wait_gathers(sv, dv, xv, bv, gs)

            if j == 0:
                @pl.when(m >= 1)
                def _():
                    wait_scatter(pxv, pdv, pss)
            else:
                wait_scatter(pxv, pdv, pss)

            @pl.when(k + 2 < NCHUNK)
            def _():
                fill_slots(k + 2, psv, pdv)
                issue_gathers(psv, pdv, pxv, pbv, pgs)

            compute(xv, bv)
            issue_scatter(xv, dv, ss)
        return 0
    lax.fori_loop(0, ntri, body, 0)

    wait_scatter(xj_c, dsl_c, ssc)

    plsc.subcore_barrier()

    _acc_rr_copy(s, lambda idx: pltpu.sync_copy(
        acc_sh.at[pl.ds(idx * CH, CH)], out_hbm.at[c, pl.ds(idx * CH, CH)]))


_edge_call = functools.partial(
    pl.kernel,
    mesh=plsc.VectorSubcoreMesh(core_axis_name="c", subcore_axis_name="s"),
    out_type=jax.ShapeDtypeStruct((NC, N, D), jnp.float32),
    scratch_types=[
        pltpu.VMEM((EPT,), jnp.int32),
        pltpu.VMEM((CH,), jnp.int32),
        pltpu.VMEM((CH,), jnp.int32),
        pltpu.VMEM((CH,), jnp.int32),
        pltpu.VMEM((CH,), jnp.int32),
        pltpu.VMEM((CH,), jnp.int32),
        pltpu.VMEM((CH,), jnp.int32),
        pltpu.VMEM((CH, D), jnp.float32),
        pltpu.VMEM((CH, D), jnp.int32),
        pltpu.VMEM((CH, D), jnp.float32),
        pltpu.VMEM((CH, D), jnp.int32),
        pltpu.VMEM((CH, D), jnp.float32),
        pltpu.VMEM((CH, D), jnp.int32),
        pltpu.VMEM_SHARED((N, D), jnp.float32),
        pltpu.SemaphoreType.DMA,
        pltpu.SemaphoreType.DMA,
        pltpu.SemaphoreType.DMA,
        pltpu.SemaphoreType.DMA,
        pltpu.SemaphoreType.DMA,
        pltpu.SemaphoreType.DMA,
    ],
)(_edge_body)


def _deg_body(dstr_hbm, out_hbm, dst_full, ones_v, acc_sh, sem):
    c = lax.axis_index("c")
    s = lax.axis_index("s")
    wid = s * NC + c

    pltpu.sync_copy(dstr_hbm.at[wid], dst_full)

    _zero_rows(ones_v, CH)
    _acc_rr_copy(s, lambda idx: pltpu.sync_copy(
        ones_v, acc_sh.at[pl.ds(idx * CH, CH)]))

    def orow(i, _):
        for j in range(D // 16):
            ones_v[i, pl.ds(j * 16, 16)] = jnp.ones((16,), jnp.float32)
        return 0
    lax.fori_loop(0, CH, orow, 0)
    plsc.subcore_barrier()

    LAG = 8

    def chunk(k, _):
        pltpu.async_copy(ones_v, acc_sh.at[dst_full.at[k]], sem, add=True)

        @pl.when(k >= LAG)
        def _():
            pltpu.make_async_copy(ones_v, acc_sh.at[dst_full.at[k - LAG]],
                                  sem).wait()
        return 0
    lax.fori_loop(0, NCHUNK, chunk, 0)
    for i in range(LAG):
        pltpu.make_async_copy(ones_v, acc_sh.at[dst_full.at[NCHUNK - LAG + i]],
                              sem).wait()

    plsc.subcore_barrier()

    _acc_rr_copy(s, lambda idx: pltpu.sync_copy(
        acc_sh.at[pl.ds(idx * CH, CH)], out_hbm.at[c, pl.ds(idx * CH, CH)]))


_deg_call = functools.partial(
    pl.kernel,
    mesh=plsc.VectorSubcoreMesh(core_axis_name="c", subcore_axis_name="s"),
    out_type=jax.ShapeDtypeStruct((NC, N, D), jnp.float32),
    scratch_types=[
        pltpu.VMEM((NCHUNK, CH), jnp.int32),
        pltpu.VMEM((CH, D), jnp.float32),
        pltpu.VMEM_SHARED((N, D), jnp.float32),
        pltpu.SemaphoreType.DMA,
    ],
)(_deg_body)


def _pack_half(t):
    u = lax.bitcast_convert_type(t.astype(jnp.bfloat16), jnp.uint16)
    u = u.astype(jnp.uint32)
    return lax.bitcast_convert_type(u[:, :64] | (u[:, 64:] << 16), jnp.int32)


def _dense(h, Wl, Wf, bf, Wls, Wfs):
    xl = jnp.dot(h, Wl, preferred_element_type=jnp.float32)
    bg = jnp.dot(h, Wf, preferred_element_type=jnp.float32) + bf
    bgs = jnp.dot(h, Wfs, preferred_element_type=jnp.float32)
    lin = jnp.dot(h, Wls, preferred_element_type=jnp.float32)
    out = jnp.maximum(bgs[:, D:] * lin + bgs[:, :D], 0.0)
    bgp = jnp.concatenate([_pack_half(bg[:, :D]), _pack_half(bg[:, D:])],
                          axis=1)
    return xl, bgp, out


def _pre_body(x_ref, wl_ref, wf_ref, bf_ref, wls_ref, wfs_ref,
              xl_ref, bg_ref, out_ref):
    xl, bg, out = _dense(x_ref[...], wl_ref[...], wf_ref[...], bf_ref[...],
                         wls_ref[...], wfs_ref[...])
    xl_ref[...] = xl
    bg_ref[...] = bg
    out_ref[...] = out


def _combine(agg_ref, deg_ref, outp_ref):
    a = agg_ref[0] + agg_ref[1]
    deg = deg_ref[0, :, :1] + deg_ref[1, :, :1]
    inv = 1.0 / jnp.maximum(deg, 1.0)
    return jnp.maximum(outp_ref[...] + a * inv, 0.0)


def _mid_body(agg_ref, deg_ref, outp_ref, wl_ref, wf_ref, bf_ref, wls_ref,
              wfs_ref, xl_ref, bg_ref, out_ref):
    h = _combine(agg_ref, deg_ref, outp_ref)
    xl, bg, out = _dense(h, wl_ref[...], wf_ref[...], bf_ref[...],
                         wls_ref[...], wfs_ref[...])
    xl_ref[...] = xl
    bg_ref[...] = bg
    out_ref[...] = out


def _post_body(agg_ref, deg_ref, outp_ref, y_ref):
    h = _combine(agg_ref, deg_ref, outp_ref)
    m = jnp.max(h, axis=-1, keepdims=True)
    ex = jnp.exp(h - m)
    y_ref[...] = h - (jnp.log(jnp.sum(ex, axis=-1, keepdims=True)) + m)


def _row_spec(w):
    return pl.BlockSpec((RB, w), lambda i: (i, 0))


def _full_spec(shape):
    nd = len(shape)
    return pl.BlockSpec(shape, lambda i: (0,) * nd)


_W_SPECS = [
    _full_spec((D, D)),
    _full_spec((D, 2 * D)),
    _full_spec((1, 2 * D)),
    _full_spec((D, D)),
    _full_spec((D, 2 * D)),
]

_DENSE_OUT = [
    jax.ShapeDtypeStruct((N, D), jnp.float32),
    jax.ShapeDtypeStruct((N, D), jnp.int32),
    jax.ShapeDtypeStruct((N, D), jnp.float32),
]

_pre_call = pl.pallas_call(
    _pre_body,
    grid=(GRID,),
    in_specs=[_row_spec(D)] + _W_SPECS,
    out_specs=[_row_spec(D), _row_spec(D), _row_spec(D)],
    out_shape=_DENSE_OUT,
)

_AGG_SPEC = pl.BlockSpec((NC, RB, D), lambda i: (0, i, 0))

_mid_call = pl.pallas_call(
    _mid_body,
    grid=(GRID,),
    in_specs=[_AGG_SPEC, _AGG_SPEC, _row_spec(D)] + _W_SPECS,
    out_specs=[_row_spec(D), _row_spec(D), _row_spec(D)],
    out_shape=_DENSE_OUT,
)

_post_call = pl.pallas_call(
    _post_body,
    grid=(GRID,),
    in_specs=[_AGG_SPEC, _AGG_SPEC, _row_spec(D)],
    out_specs=_row_spec(D),
    out_shape=jax.ShapeDtypeStruct((N, D), jnp.float32),
)


def kernel(x, edge_index, W_lin, W_film, b_film, W_skip, W_film_skip):
    packed = (edge_index[0] | (edge_index[1] << 16)).reshape(NW, EPT)

    deg2 = _deg_call(edge_index[1].reshape(NW, NCHUNK, CH))
    xl, bg, out = _pre_call(x, W_lin[0], W_film[0], b_film[0].reshape(1, -1),
                            W_skip[0], W_film_skip[0])
    agg = _edge_call(xl, bg, packed)
    for l in range(1, 3):
        xl, bg, out = _mid_call(agg, deg2, out, W_lin[l], W_film[l],
                                b_film[l].reshape(1, -1), W_skip[l],
                                W_film_skip[l])
        agg = _edge_call(xl, bg, packed)
    return _post_call(agg, deg2, out)
